# Initial kernel scaffold; baseline (speedup 1.0000x reference)
import jax
import jax.numpy as jnp
from jax.experimental import pallas as pl


def kernel(x, embedding):
    raise NotImplementedError("write your pallas kernel here")


# speedup vs baseline: 3.1842x; 3.1842x over previous
import jax
import jax.numpy as jnp
from jax.experimental import pallas as pl


def _copy_body(emb_ref, out_ref):
    out_ref[0, :, :] = emb_ref[:, :]


def kernel(x, embedding):
    seq_len = x.shape[1]
    max_len, d_model = embedding.shape
    block_rows = 1024
    grid = (seq_len // block_rows,)
    out = pl.pallas_call(
        _copy_body,
        grid=grid,
        in_specs=[pl.BlockSpec((block_rows, d_model), lambda i: (i, 0))],
        out_specs=pl.BlockSpec((1, block_rows, d_model), lambda i: (0, i, 0)),
        out_shape=jax.ShapeDtypeStruct((1, seq_len, d_model), jnp.float32),
    )(embedding)
    return out


# speedup vs baseline: 3.4074x; 1.0701x over previous
import jax
import jax.numpy as jnp
from jax.experimental import pallas as pl


def _copy_body(emb_ref, out_ref):
    out_ref[0, :, :] = emb_ref[:, :]


def kernel(x, embedding):
    seq_len = x.shape[1]
    max_len, d_model = embedding.shape
    block_rows = 2048
    grid = (seq_len // block_rows,)
    out = pl.pallas_call(
        _copy_body,
        grid=grid,
        in_specs=[pl.BlockSpec((block_rows, d_model), lambda i: (i, 0))],
        out_specs=pl.BlockSpec((1, block_rows, d_model), lambda i: (0, i, 0)),
        out_shape=jax.ShapeDtypeStruct((1, seq_len, d_model), jnp.float32),
    )(embedding)
    return out


# speedup vs baseline: 3.6345x; 1.0667x over previous
import jax
import jax.numpy as jnp
from jax.experimental import pallas as pl


def _copy_body(emb_ref, out_ref):
    out_ref[0, :, :] = emb_ref[:, :]


def kernel(x, embedding):
    seq_len = x.shape[1]
    max_len, d_model = embedding.shape
    block_rows = 4096
    grid = (seq_len // block_rows,)
    out = pl.pallas_call(
        _copy_body,
        grid=grid,
        in_specs=[pl.BlockSpec((block_rows, d_model), lambda i: (i, 0))],
        out_specs=pl.BlockSpec((1, block_rows, d_model), lambda i: (0, i, 0)),
        out_shape=jax.ShapeDtypeStruct((1, seq_len, d_model), jnp.float32),
    )(embedding)
    return out
